# baseline (speedup 1.0000x reference)
import functools

import jax
import jax.numpy as jnp
from jax import lax
from jax.experimental import pallas as pl
from jax.experimental.pallas import tpu as pltpu
from jax.experimental.pallas import tpu_sc as plsc

DIM = 64
LANES = 16
ICH = 4096
UNROLL = 16


def _make_kernel(B, V):
    info = plsc.get_sparse_core_info()
    NW = info.num_cores * info.num_subcores
    rows_per_w = DIM // NW
    n_ich = B // ICH
    n_units = rows_per_w * 3
    mesh = plsc.VectorSubcoreMesh(core_axis_name="c", subcore_axis_name="s")

    @functools.partial(
        pl.kernel,
        mesh=mesh,
        out_type=jax.ShapeDtypeStruct((DIM, B), jnp.float32),
        compiler_params=pltpu.CompilerParams(needs_layout_passes=False),
        scratch_types=[
            pltpu.VMEM((V,), jnp.float32),
            pltpu.VMEM((B,), jnp.float32),
            pltpu.VMEM((ICH,), jnp.int32),
            pltpu.VMEM((ICH,), jnp.int32),
            pltpu.SemaphoreType.DMA,
            pltpu.SemaphoreType.DMA,
            pltpu.SemaphoreType.DMA,
            pltpu.SemaphoreType.DMA,
        ],
    )
    def k(uid_hbm, iid_hbm, cid_hbm, tu_hbm, ti_hbm, tc_hbm, out_hbm,
          rowbuf, acc, ib0, ib1, s0, si0, si1, so):
        wid = lax.axis_index("s") * info.num_cores + lax.axis_index("c")
        tabs = (tu_hbm, ti_hbm, tc_hbm)
        idxs = (uid_hbm, iid_hbm, cid_hbm)
        ibufs = (ib0, ib1)
        isems = (si0, si1)

        def start_stream(u):
            r, t = u // 3, u % 3
            j = wid * rows_per_w + r
            return [pltpu.async_copy(tabs[t].at[j], rowbuf, s0)]

        def gather_pass(u):
            r, t = u // 3, u % 3
            first = t == 0
            idesc = [None, None]
            idesc[0] = pltpu.async_copy(idxs[t].at[pl.ds(0, ICH)], ibufs[0], isems[0])
            for c in range(n_ich):
                if c + 1 < n_ich:
                    nb = (c + 1) % 2
                    idesc[nb] = pltpu.async_copy(
                        idxs[t].at[pl.ds((c + 1) * ICH, ICH)], ibufs[nb], isems[nb])
                idesc[c % 2].wait()
                ib = ibufs[c % 2]

                def body(v, carry, _c=c, _ib=ib, _first=first):
                    for s in range(UNROLL):
                        off = v * LANES * UNROLL + s * LANES
                        iv = _ib[pl.ds(off, LANES)]
                        g = plsc.load_gather(rowbuf, [iv])
                        sl = pl.ds(_c * ICH + off, LANES)
                        if _first:
                            acc[sl] = g
                        else:
                            plsc.addupdate(acc.at[sl], g)
                    return carry

                lax.fori_loop(0, ICH // (LANES * UNROLL), body, 0)

        out_desc = None
        descs = start_stream(0)
        for u in range(n_units):
            for d in descs:
                d.wait()
            if u == 3 and out_desc is not None:
                out_desc.wait()
            gather_pass(u)
            if u + 1 < n_units:
                descs = start_stream(u + 1)
            if u == 2:
                out_desc = pltpu.async_copy(acc, out_hbm.at[wid * rows_per_w], so)
        pltpu.sync_copy(acc, out_hbm.at[wid * rows_per_w + 1])

    return k


def kernel(user_id, item_id, context_id, table_user, table_item, table_context, batch_size):
    B = user_id.shape[0]
    V = table_user.shape[0]
    k = _make_kernel(B, V)
    out_t = k(user_id, item_id, context_id,
              table_user.T, table_item.T, table_context.T)
    return out_t.T


# speedup vs baseline: 1.0403x; 1.0403x over previous
import functools

import jax
import jax.numpy as jnp
from jax import lax
from jax.experimental import pallas as pl
from jax.experimental.pallas import tpu as pltpu
from jax.experimental.pallas import tpu_sc as plsc

DIM = 64
LANES = 16
ICH = 4096
UNROLL = 8


def _make_kernel(B, V):
    info = plsc.get_sparse_core_info()
    NW = info.num_cores * info.num_subcores
    rows_per_w = DIM // NW
    n_ich = B // ICH
    n_units = rows_per_w * 3
    mesh = plsc.VectorSubcoreMesh(core_axis_name="c", subcore_axis_name="s")

    @functools.partial(
        pl.kernel,
        mesh=mesh,
        out_type=jax.ShapeDtypeStruct((DIM, B), jnp.float32),
        compiler_params=pltpu.CompilerParams(needs_layout_passes=False),
        scratch_types=[
            pltpu.VMEM((V,), jnp.float32),
            pltpu.VMEM((B,), jnp.float32),
            pltpu.VMEM((ICH,), jnp.int32),
            pltpu.VMEM((ICH,), jnp.int32),
            pltpu.SemaphoreType.DMA,
            pltpu.SemaphoreType.DMA,
            pltpu.SemaphoreType.DMA,
            pltpu.SemaphoreType.DMA,
        ],
    )
    def k(uid_hbm, iid_hbm, cid_hbm, tu_hbm, ti_hbm, tc_hbm, out_hbm,
          rowbuf, acc, ib0, ib1, s0, si0, si1, so):
        wid = lax.axis_index("s") * info.num_cores + lax.axis_index("c")
        tabs = (tu_hbm, ti_hbm, tc_hbm)
        idxs = (uid_hbm, iid_hbm, cid_hbm)
        ibufs = (ib0, ib1)
        isems = (si0, si1)

        def start_stream(u):
            r, t = u // 3, u % 3
            j = wid * rows_per_w + r
            return [pltpu.async_copy(tabs[t].at[j], rowbuf, s0)]

        def gather_pass(u):
            r, t = u // 3, u % 3
            first = t == 0
            idesc = [None, None]
            idesc[0] = pltpu.async_copy(idxs[t].at[pl.ds(0, ICH)], ibufs[0], isems[0])
            for c in range(n_ich):
                if c + 1 < n_ich:
                    nb = (c + 1) % 2
                    idesc[nb] = pltpu.async_copy(
                        idxs[t].at[pl.ds((c + 1) * ICH, ICH)], ibufs[nb], isems[nb])
                idesc[c % 2].wait()
                ib = ibufs[c % 2]

                def body(v, carry, _c=c, _ib=ib, _first=first):
                    for s in range(UNROLL):
                        off = v * LANES * UNROLL + s * LANES
                        iv = _ib[pl.ds(off, LANES)]
                        g = plsc.load_gather(rowbuf, [iv])
                        sl = pl.ds(_c * ICH + off, LANES)
                        if _first:
                            acc[sl] = g
                        else:
                            plsc.addupdate(acc.at[sl], g)
                    return carry

                lax.fori_loop(0, ICH // (LANES * UNROLL), body, 0)

        out_desc = None
        descs = start_stream(0)
        for u in range(n_units):
            for d in descs:
                d.wait()
            if u == 3 and out_desc is not None:
                out_desc.wait()
            gather_pass(u)
            if u + 1 < n_units:
                descs = start_stream(u + 1)
            if u == 2:
                out_desc = pltpu.async_copy(acc, out_hbm.at[wid * rows_per_w], so)
        pltpu.sync_copy(acc, out_hbm.at[wid * rows_per_w + 1])

    return k


def kernel(user_id, item_id, context_id, table_user, table_item, table_context, batch_size):
    B = user_id.shape[0]
    V = table_user.shape[0]
    k = _make_kernel(B, V)
    out_t = k(user_id, item_id, context_id,
              table_user.T, table_item.T, table_context.T)
    return out_t.T


# speedup vs baseline: 1.0404x; 1.0001x over previous
import functools

import jax
import jax.numpy as jnp
from jax import lax
from jax.experimental import pallas as pl
from jax.experimental.pallas import tpu as pltpu
from jax.experimental.pallas import tpu_sc as plsc

DIM = 64
LANES = 16
ICH = 4096
UNROLL = 8


def _make_kernel(B, V):
    info = plsc.get_sparse_core_info()
    NW = info.num_cores * info.num_subcores
    rows_per_w = DIM // NW
    n_ich = B // ICH
    n_units = rows_per_w * 3
    mesh = plsc.VectorSubcoreMesh(core_axis_name="c", subcore_axis_name="s")

    @functools.partial(
        pl.kernel,
        mesh=mesh,
        out_type=jax.ShapeDtypeStruct((DIM, B), jnp.float32),
        compiler_params=pltpu.CompilerParams(
            needs_layout_passes=False,
            skip_device_barrier=True,
            disable_semaphore_checks=True,
        ),
        scratch_types=[
            pltpu.VMEM((V,), jnp.float32),
            pltpu.VMEM((B,), jnp.float32),
            pltpu.VMEM((ICH,), jnp.int32),
            pltpu.VMEM((ICH,), jnp.int32),
            pltpu.SemaphoreType.DMA,
            pltpu.SemaphoreType.DMA,
            pltpu.SemaphoreType.DMA,
            pltpu.SemaphoreType.DMA,
        ],
    )
    def k(uid_hbm, iid_hbm, cid_hbm, tu_hbm, ti_hbm, tc_hbm, out_hbm,
          rowbuf, acc, ib0, ib1, s0, si0, si1, so):
        wid = lax.axis_index("s") * info.num_cores + lax.axis_index("c")
        tabs = (tu_hbm, ti_hbm, tc_hbm)
        idxs = (uid_hbm, iid_hbm, cid_hbm)
        ibufs = (ib0, ib1)
        isems = (si0, si1)

        def start_stream(u):
            r, t = u // 3, u % 3
            j = wid * rows_per_w + r
            return [pltpu.async_copy(tabs[t].at[j], rowbuf, s0)]

        def gather_pass(u):
            r, t = u // 3, u % 3
            first = t == 0
            idesc = [None, None]
            idesc[0] = pltpu.async_copy(idxs[t].at[pl.ds(0, ICH)], ibufs[0], isems[0])
            for c in range(n_ich):
                if c + 1 < n_ich:
                    nb = (c + 1) % 2
                    idesc[nb] = pltpu.async_copy(
                        idxs[t].at[pl.ds((c + 1) * ICH, ICH)], ibufs[nb], isems[nb])
                idesc[c % 2].wait()
                ib = ibufs[c % 2]

                def body(v, carry, _c=c, _ib=ib, _first=first):
                    for s in range(UNROLL):
                        off = v * LANES * UNROLL + s * LANES
                        iv = _ib[pl.ds(off, LANES)]
                        g = plsc.load_gather(rowbuf, [iv])
                        sl = pl.ds(_c * ICH + off, LANES)
                        if _first:
                            acc[sl] = g
                        else:
                            plsc.addupdate(acc.at[sl], g)
                    return carry

                lax.fori_loop(0, ICH // (LANES * UNROLL), body, 0)

        out_desc = None
        descs = start_stream(0)
        for u in range(n_units):
            for d in descs:
                d.wait()
            if u == 3 and out_desc is not None:
                out_desc.wait()
            gather_pass(u)
            if u + 1 < n_units:
                descs = start_stream(u + 1)
            if u == 2:
                out_desc = pltpu.async_copy(acc, out_hbm.at[wid * rows_per_w], so)
        pltpu.sync_copy(acc, out_hbm.at[wid * rows_per_w + 1])

    return k


def kernel(user_id, item_id, context_id, table_user, table_item, table_context, batch_size):
    B = user_id.shape[0]
    V = table_user.shape[0]
    k = _make_kernel(B, V)
    out_t = k(user_id, item_id, context_id,
              table_user.T, table_item.T, table_context.T)
    return out_t.T


# speedup vs baseline: 1.0926x; 1.0501x over previous
import functools

import jax
import jax.numpy as jnp
from jax import lax
from jax.experimental import pallas as pl
from jax.experimental.pallas import tpu as pltpu
from jax.experimental.pallas import tpu_sc as plsc

DIM = 64
LANES = 16
ICH = 4096
UNROLL = 8


def _make_kernel(B, V):
    info = plsc.get_sparse_core_info()
    NW = info.num_cores * info.num_subcores
    rows_per_w = DIM // NW
    n_ich = B // ICH
    n_units = rows_per_w * 3
    mesh = plsc.VectorSubcoreMesh(core_axis_name="c", subcore_axis_name="s")

    @functools.partial(
        pl.kernel,
        mesh=mesh,
        out_type=jax.ShapeDtypeStruct((DIM, B), jnp.float32),
        compiler_params=pltpu.CompilerParams(needs_layout_passes=False),
        scratch_types=[
            pltpu.VMEM((V,), jnp.float32),
            pltpu.VMEM((B,), jnp.float32),
            pltpu.VMEM((ICH,), jnp.int32),
            pltpu.VMEM((ICH,), jnp.int32),
            pltpu.SemaphoreType.DMA,
            pltpu.SemaphoreType.DMA,
            pltpu.SemaphoreType.DMA,
            pltpu.SemaphoreType.DMA,
        ],
    )
    def k(uid_hbm, iid_hbm, cid_hbm, tu_hbm, ti_hbm, tc_hbm, out_hbm,
          rowbuf, acc, ib0, ib1, s0, si0, si1, so):
        wid = lax.axis_index("s") * info.num_cores + lax.axis_index("c")
        tabs = (tu_hbm, ti_hbm, tc_hbm)
        idxs = (uid_hbm, iid_hbm, cid_hbm)
        ibufs = (ib0, ib1)
        isems = (si0, si1)

        def start_stream(u):
            r, t = u // 3, u % 3
            j = wid * rows_per_w + r
            return [pltpu.async_copy(tabs[t].at[j], rowbuf, s0)]

        def gather_pass(u, first_idesc):
            r, t = u // 3, u % 3
            first = t == 0
            idesc = [None, None]
            idesc[0] = first_idesc
            nxt = None
            for c in range(n_ich):
                nb = (c + 1) % 2
                if c + 1 < n_ich:
                    idesc[nb] = pltpu.async_copy(
                        idxs[t].at[pl.ds((c + 1) * ICH, ICH)], ibufs[nb], isems[nb])
                elif u + 1 < n_units:
                    tn = (u + 1) % 3
                    nxt = pltpu.async_copy(
                        idxs[tn].at[pl.ds(0, ICH)], ibufs[nb], isems[nb])
                idesc[c % 2].wait()
                ib = ibufs[c % 2]

                def body(v, carry, _c=c, _ib=ib, _first=first):
                    for s in range(UNROLL):
                        off = v * LANES * UNROLL + s * LANES
                        iv = _ib[pl.ds(off, LANES)]
                        g = plsc.load_gather(rowbuf, [iv])
                        sl = pl.ds(_c * ICH + off, LANES)
                        if _first:
                            acc[sl] = g
                        else:
                            plsc.addupdate(acc.at[sl], g)
                    return carry

                lax.fori_loop(0, ICH // (LANES * UNROLL), body, 0)
            return nxt

        out_desc = None
        descs = start_stream(0)
        nxt_idesc = pltpu.async_copy(idxs[0].at[pl.ds(0, ICH)], ibufs[0], isems[0])
        for u in range(n_units):
            for d in descs:
                d.wait()
            if u == 3 and out_desc is not None:
                out_desc.wait()
            nxt_idesc = gather_pass(u, nxt_idesc)
            if u + 1 < n_units:
                descs = start_stream(u + 1)
            if u == 2:
                out_desc = pltpu.async_copy(acc, out_hbm.at[wid * rows_per_w], so)
        pltpu.sync_copy(acc, out_hbm.at[wid * rows_per_w + 1])

    return k


def kernel(user_id, item_id, context_id, table_user, table_item, table_context, batch_size):
    B = user_id.shape[0]
    V = table_user.shape[0]
    k = _make_kernel(B, V)
    out_t = k(user_id, item_id, context_id,
              table_user.T, table_item.T, table_context.T)
    return out_t.T
